# Initial kernel scaffold; baseline (speedup 1.0000x reference)
import jax
import jax.numpy as jnp
from jax.experimental import pallas as pl


def kernel(inp, W, b):
    raise NotImplementedError("write your pallas kernel here")


# speedup vs baseline: 1.2303x; 1.2303x over previous
import functools

import jax
import jax.numpy as jnp
from jax.experimental import pallas as pl

D_MODEL_K = 768
N_EXP_K = 64
TOPK_K = 2


def _fused_body(x_ref, wt_ref, b_ref, i1_ref, i2_ref, s1_ref, s2_ref):
    x = x_ref[...]
    wt = wt_ref[...]
    gate = jnp.dot(x, wt, preferred_element_type=jnp.float32) + b_ref[...]
    bt = gate.shape[0]
    lane = jax.lax.broadcasted_iota(jnp.int32, (bt, N_EXP_K), 1)

    m1 = jnp.max(gate, axis=1, keepdims=True)
    i1 = jnp.min(jnp.where(gate == m1, lane, N_EXP_K), axis=1, keepdims=True)
    gate2 = jnp.where(lane == i1, -jnp.inf, gate)
    m2 = jnp.max(gate2, axis=1, keepdims=True)
    i2 = jnp.min(jnp.where(gate2 == m2, lane, N_EXP_K), axis=1, keepdims=True)

    e2 = jnp.exp(m2 - m1)
    denom = 1.0 + e2
    i1_ref[...] = i1
    i2_ref[...] = i2
    s1_ref[...] = 1.0 / denom
    s2_ref[...] = e2 / denom


@functools.partial(jax.jit, static_argnames=("bt",))
def _fused(inp, wt, b2, bt):
    n_tok = inp.shape[0]
    grid = (n_tok // bt,)
    out_shapes = (
        jax.ShapeDtypeStruct((n_tok, 1), jnp.int32),
        jax.ShapeDtypeStruct((n_tok, 1), jnp.int32),
        jax.ShapeDtypeStruct((n_tok, 1), jnp.float32),
        jax.ShapeDtypeStruct((n_tok, 1), jnp.float32),
    )
    os_ = pl.BlockSpec((bt, 1), lambda i: (i, 0))
    return pl.pallas_call(
        _fused_body,
        grid=grid,
        in_specs=[
            pl.BlockSpec((bt, D_MODEL_K), lambda i: (i, 0)),
            pl.BlockSpec((D_MODEL_K, N_EXP_K), lambda i: (0, 0)),
            pl.BlockSpec((1, N_EXP_K), lambda i: (0, 0)),
        ],
        out_specs=(os_, os_, os_, os_),
        out_shape=out_shapes,
    )(inp, wt, b2)


def kernel(inp, W, b):
    n_tok = inp.shape[0]
    i1, i2, s1, s2 = _fused(inp, W.T, b[None, :], 512)
    idx = jnp.concatenate([i1, i2], axis=1).reshape(-1)
    score = jnp.concatenate([s1, s2], axis=1).reshape(n_tok, 1, TOPK_K)
    return (idx, score)


# speedup vs baseline: 1.2961x; 1.0535x over previous
import functools

import jax
import jax.numpy as jnp
from jax.experimental import pallas as pl

D_MODEL_K = 768
N_EXP_K = 64
TOPK_K = 2


def _fused_body(x_ref, wt_ref, b_ref, i1_ref, i2_ref, s1_ref, s2_ref):
    x = x_ref[...]
    wt = wt_ref[...]
    gate = jnp.dot(x, wt, preferred_element_type=jnp.float32) + b_ref[...]
    bt = gate.shape[0]
    lane = jax.lax.broadcasted_iota(jnp.int32, (bt, N_EXP_K), 1).astype(jnp.float32)

    m1 = jnp.max(gate, axis=1, keepdims=True)
    i1 = jnp.min(jnp.where(gate == m1, lane, float(N_EXP_K)), axis=1, keepdims=True)
    gate2 = jnp.where(lane == i1, -jnp.inf, gate)
    m2 = jnp.max(gate2, axis=1, keepdims=True)
    i2 = jnp.min(jnp.where(gate2 == m2, lane, float(N_EXP_K)), axis=1, keepdims=True)

    e2 = jnp.exp(m2 - m1)
    denom = 1.0 + e2
    i1_ref[...] = i1.astype(jnp.int32)
    i2_ref[...] = i2.astype(jnp.int32)
    s1_ref[...] = 1.0 / denom
    s2_ref[...] = e2 / denom


@functools.partial(jax.jit, static_argnames=("bt",))
def _fused(inp, wt, b2, bt):
    n_tok = inp.shape[0]
    grid = (n_tok // bt,)
    out_shapes = (
        jax.ShapeDtypeStruct((n_tok, 1), jnp.int32),
        jax.ShapeDtypeStruct((n_tok, 1), jnp.int32),
        jax.ShapeDtypeStruct((n_tok, 1), jnp.float32),
        jax.ShapeDtypeStruct((n_tok, 1), jnp.float32),
    )
    os_ = pl.BlockSpec((bt, 1), lambda i: (i, 0))
    return pl.pallas_call(
        _fused_body,
        grid=grid,
        in_specs=[
            pl.BlockSpec((bt, D_MODEL_K), lambda i: (i, 0)),
            pl.BlockSpec((D_MODEL_K, N_EXP_K), lambda i: (0, 0)),
            pl.BlockSpec((1, N_EXP_K), lambda i: (0, 0)),
        ],
        out_specs=(os_, os_, os_, os_),
        out_shape=out_shapes,
    )(inp, wt, b2)


def kernel(inp, W, b):
    n_tok = inp.shape[0]
    i1, i2, s1, s2 = _fused(inp, W.T, b[None, :], 512)
    idx = jnp.concatenate([i1, i2], axis=1).reshape(-1)
    score = jnp.concatenate([s1, s2], axis=1).reshape(n_tok, 1, TOPK_K)
    return (idx, score)


# speedup vs baseline: 1.6109x; 1.2428x over previous
import functools

import jax
import jax.numpy as jnp
from jax.experimental import pallas as pl

D_MODEL_K = 768
N_EXP_K = 64
TOPK_K = 2


def _fused_body(x_ref, wt_ref, b_ref, i1_ref, i2_ref, s1_ref, s2_ref):
    x = x_ref[...]
    wt = wt_ref[...]
    gate = jnp.dot(x, wt, preferred_element_type=jnp.float32) + b_ref[...]
    bt = gate.shape[0]
    lane = jax.lax.broadcasted_iota(jnp.int32, (bt, N_EXP_K), 1).astype(jnp.float32)

    m1 = jnp.max(gate, axis=1, keepdims=True)
    i1 = jnp.min(jnp.where(gate == m1, lane, float(N_EXP_K)), axis=1, keepdims=True)
    gate2 = jnp.where(lane == i1, -jnp.inf, gate)
    m2 = jnp.max(gate2, axis=1, keepdims=True)
    i2 = jnp.min(jnp.where(gate2 == m2, lane, float(N_EXP_K)), axis=1, keepdims=True)

    e2 = jnp.exp(m2 - m1)
    denom = 1.0 + e2
    i1_ref[...] = i1.astype(jnp.int32)
    i2_ref[...] = i2.astype(jnp.int32)
    s1_ref[...] = 1.0 / denom
    s2_ref[...] = e2 / denom


@functools.partial(jax.jit, static_argnames=("bt",))
def _fused(inp, wt, b2, bt):
    n_tok = inp.shape[0]
    grid = (n_tok // bt,)
    out_shapes = (
        jax.ShapeDtypeStruct((n_tok, 1), jnp.int32),
        jax.ShapeDtypeStruct((n_tok, 1), jnp.int32),
        jax.ShapeDtypeStruct((n_tok, 1), jnp.float32),
        jax.ShapeDtypeStruct((n_tok, 1), jnp.float32),
    )
    os_ = pl.BlockSpec((bt, 1), lambda i: (i, 0))
    return pl.pallas_call(
        _fused_body,
        grid=grid,
        in_specs=[
            pl.BlockSpec((bt, D_MODEL_K), lambda i: (i, 0)),
            pl.BlockSpec((D_MODEL_K, N_EXP_K), lambda i: (0, 0)),
            pl.BlockSpec((1, N_EXP_K), lambda i: (0, 0)),
        ],
        out_specs=(os_, os_, os_, os_),
        out_shape=out_shapes,
    )(inp, wt, b2)


def kernel(inp, W, b):
    n_tok = inp.shape[0]
    i1, i2, s1, s2 = _fused(inp, W.T, b[None, :], 2048)
    idx = jnp.concatenate([i1, i2], axis=1).reshape(-1)
    score = jnp.concatenate([s1, s2], axis=1).reshape(n_tok, 1, TOPK_K)
    return (idx, score)


# speedup vs baseline: 1.6847x; 1.0458x over previous
import functools

import jax
import jax.numpy as jnp
from jax.experimental import pallas as pl

D_MODEL_K = 768
N_EXP_K = 64
TOPK_K = 2


def _fused_body(x_ref, wt_ref, b_ref, i1_ref, i2_ref, s1_ref, s2_ref):
    x = x_ref[...]
    wt = wt_ref[...]
    gate = jnp.dot(x, wt, preferred_element_type=jnp.float32) + b_ref[...]
    bt = gate.shape[0]
    lane = jax.lax.broadcasted_iota(jnp.int32, (bt, N_EXP_K), 1).astype(jnp.float32)

    m1 = jnp.max(gate, axis=1, keepdims=True)
    i1 = jnp.min(jnp.where(gate == m1, lane, float(N_EXP_K)), axis=1, keepdims=True)
    gate2 = jnp.where(lane == i1, -jnp.inf, gate)
    m2 = jnp.max(gate2, axis=1, keepdims=True)
    i2 = jnp.min(jnp.where(gate2 == m2, lane, float(N_EXP_K)), axis=1, keepdims=True)

    e2 = jnp.exp(m2 - m1)
    denom = 1.0 + e2
    i1_ref[...] = i1.astype(jnp.int32)
    i2_ref[...] = i2.astype(jnp.int32)
    s1_ref[...] = 1.0 / denom
    s2_ref[...] = e2 / denom


@functools.partial(jax.jit, static_argnames=("bt",))
def _fused(inp, wt, b2, bt):
    n_tok = inp.shape[0]
    grid = (n_tok // bt,)
    out_shapes = (
        jax.ShapeDtypeStruct((n_tok, 1), jnp.int32),
        jax.ShapeDtypeStruct((n_tok, 1), jnp.int32),
        jax.ShapeDtypeStruct((n_tok, 1), jnp.float32),
        jax.ShapeDtypeStruct((n_tok, 1), jnp.float32),
    )
    os_ = pl.BlockSpec((bt, 1), lambda i: (i, 0))
    return pl.pallas_call(
        _fused_body,
        grid=grid,
        in_specs=[
            pl.BlockSpec((bt, D_MODEL_K), lambda i: (i, 0)),
            pl.BlockSpec((D_MODEL_K, N_EXP_K), lambda i: (0, 0)),
            pl.BlockSpec((1, N_EXP_K), lambda i: (0, 0)),
        ],
        out_specs=(os_, os_, os_, os_),
        out_shape=out_shapes,
    )(inp, wt, b2)


def kernel(inp, W, b):
    n_tok = inp.shape[0]
    i1, i2, s1, s2 = _fused(inp, W.T, b[None, :], 4096)
    idx = jnp.concatenate([i1, i2], axis=1).reshape(-1)
    score = jnp.concatenate([s1, s2], axis=1).reshape(n_tok, 1, TOPK_K)
    return (idx, score)
